# Initial kernel scaffold; baseline (speedup 1.0000x reference)
import jax
import jax.numpy as jnp
from jax.experimental import pallas as pl


def kernel(loc_data, conf_data, priors, targets):
    raise NotImplementedError("write your pallas kernel here")


# speedup vs baseline: 6.1885x; 6.1885x over previous
import functools

import jax
import jax.numpy as jnp
from jax.experimental import pallas as pl

_VARIANCES = (0.1, 0.2)
_THRESHOLD = 0.5
_NEGPOS_RATIO = 3

_NP = 8732
_R, _C = 72, 128
_PAD = _R * _C - _NP


@jax.jit
def kernel(loc_data, conf_data, priors, targets):
    num, num_priors, num_classes = conf_data.shape
    num_objs = targets.shape[1]

    conf_p = jnp.pad(jnp.transpose(conf_data, (0, 2, 1)),
                     ((0, 0), (0, 0), (0, _PAD))).reshape(num, num_classes, _R, _C)
    loc_p = jnp.pad(jnp.transpose(loc_data, (0, 2, 1)),
                    ((0, 0), (0, 0), (0, _PAD))).reshape(num, 4, _R, _C)
    priors_p = jnp.pad(priors.T, ((0, 0), (0, _PAD))).reshape(4, _R, _C)
    truths = jnp.transpose(targets[:, :, :4], (0, 2, 1))
    labels = targets[:, :, 4].reshape(num, 1, num_objs)

    body = functools.partial(_acc_body, num_objs=num_objs,
                             num_classes=num_classes, num_images=num)
    out = pl.pallas_call(
        body,
        grid=(num,),
        in_specs=[
            pl.BlockSpec((1, 4, num_objs), lambda i: (i, 0, 0)),
            pl.BlockSpec((1, 1, num_objs), lambda i: (i, 0, 0)),
            pl.BlockSpec((4, _R, _C), lambda i: (0, 0, 0)),
            pl.BlockSpec((1, 4, _R, _C), lambda i: (i, 0, 0, 0)),
            pl.BlockSpec((1, num_classes, _R, _C), lambda i: (i, 0, 0, 0)),
        ],
        out_specs=pl.BlockSpec((1, 128), lambda i: (0, 0)),
        out_shape=jax.ShapeDtypeStruct((1, 128), jnp.float32),
    )(truths, labels, priors_p, loc_p, conf_p)
    n = out[0, 2]
    return (out[0, 0] / n, out[0, 1] / n)


def _acc_body(truths_ref, labels_ref, priors_ref, loc_ref, conf_ref,
              out_ref, num_objs, num_classes, num_images):
    i = pl.program_id(0)
    loss_l_i, loss_c_i, npos = _image_losses(
        truths_ref, labels_ref, priors_ref, loc_ref, conf_ref,
        num_objs, num_classes)

    @pl.when(i == 0)
    def _():
        out_ref[...] = jnp.zeros_like(out_ref)

    lane = jax.lax.broadcasted_iota(jnp.int32, (1, 128), 1)
    add = (jnp.where(lane == 0, loss_l_i, 0.0)
           + jnp.where(lane == 1, loss_c_i, 0.0)
           + jnp.where(lane == 2, npos, 0.0))
    out_ref[...] += add


def _image_losses(truths_ref, labels_ref, priors_ref, loc_ref, conf_ref,
                  num_objs, num_classes):
    pcx = priors_ref[0]
    pcy = priors_ref[1]
    pw = priors_ref[2]
    ph = priors_ref[3]
    px1 = pcx - pw / 2.0
    py1 = pcy - ph / 2.0
    px2 = pcx + pw / 2.0
    py2 = pcy + ph / 2.0
    parea = (px2 - px1) * (py2 - py1)

    idx = (jax.lax.broadcasted_iota(jnp.int32, (_R, _C), 0) * _C
           + jax.lax.broadcasted_iota(jnp.int32, (_R, _C), 1))

    btov = jnp.full((_R, _C), -1.0, dtype=jnp.float32)
    btidx = jnp.zeros((_R, _C), dtype=jnp.int32)
    bp_list = []
    for j in range(num_objs):
        tx1 = truths_ref[0, 0, j]
        ty1 = truths_ref[0, 1, j]
        tx2 = truths_ref[0, 2, j]
        ty2 = truths_ref[0, 3, j]
        ix = jnp.clip(jnp.minimum(px2, tx2) - jnp.maximum(px1, tx1), 0.0, None)
        iy = jnp.clip(jnp.minimum(py2, ty2) - jnp.maximum(py1, ty1), 0.0, None)
        inter = ix * iy
        tarea = (tx2 - tx1) * (ty2 - ty1)
        ov = inter / (tarea + parea - inter)
        upd = ov > btov
        btov = jnp.where(upd, ov, btov)
        btidx = jnp.where(upd, j, btidx)
        m = jnp.max(ov)
        bp_list.append(jnp.min(jnp.where(ov == m, idx, jnp.int32(2**30))))

    for j in range(num_objs):
        eq = idx == bp_list[j]
        btov = jnp.where(eq, 2.0, btov)
        btidx = jnp.where(eq, j, btidx)

    conf_t = jnp.zeros((_R, _C), dtype=jnp.float32)
    for j in range(num_objs):
        conf_t = jnp.where(btidx == j, labels_ref[0, 0, j], conf_t)
    conf_t = jnp.where(btov < _THRESHOLD, 0.0, conf_t)
    pos = (conf_t > 0.0) & (idx < _NP)
    npos = jnp.sum(pos.astype(jnp.float32))

    mt = []
    for c in range(4):
        acc = jnp.zeros((_R, _C), dtype=jnp.float32)
        for j in range(num_objs):
            acc = jnp.where(btidx == j, truths_ref[0, c, j], acc)
        mt.append(acc)
    mx1, my1, mx2, my2 = mt
    g_cx = ((mx1 + mx2) / 2.0 - pcx) / (_VARIANCES[0] * pw)
    g_cy = ((my1 + my2) / 2.0 - pcy) / (_VARIANCES[0] * ph)
    g_w = jnp.log((mx2 - mx1) / pw) / _VARIANCES[1]
    g_h = jnp.log((my2 - my1) / ph) / _VARIANCES[1]
    loss_l_i = jnp.float32(0.0)
    for c, g in enumerate((g_cx, g_cy, g_w, g_h)):
        d = loc_ref[0, c] - g
        ad = jnp.abs(d)
        sl1 = jnp.where(ad < 1.0, 0.5 * d * d, ad - 0.5)
        loss_l_i += jnp.sum(jnp.where(pos, sl1, 0.0))

    x = conf_ref[0]
    m = jnp.max(x, axis=0)
    s = jnp.sum(jnp.exp(x - m[None]), axis=0)
    lse = m + jnp.log(s)
    c0 = x[0]
    c1 = x[1]
    ce_pos = jnp.sum(jnp.where(pos, lse - c1, 0.0))
    v = jnp.where(pos | (idx >= _NP), 0.0, lse - c0)
    v = jnp.maximum(v, 0.0)

    k = jnp.minimum((_NEGPOS_RATIO * npos).astype(jnp.int32),
                    jnp.int32(_NP - 1))
    vb = jax.lax.bitcast_convert_type(v, jnp.int32)
    t = jnp.int32(0)
    for b in range(30, -1, -1):
        cand = t | jnp.int32(1 << b)
        cnt = jnp.sum((vb >= cand).astype(jnp.int32))
        t = jnp.where(cnt >= k, cand, t)
    cnt_gt = jnp.sum((vb > t).astype(jnp.int32))
    sum_gt = jnp.sum(jnp.where(vb > t, v, 0.0))
    tval = jax.lax.bitcast_convert_type(t, jnp.float32)
    topk = sum_gt + (k - cnt_gt).astype(jnp.float32) * tval
    topk = jnp.where(k > 0, topk, 0.0)
    loss_c_i = ce_pos + topk
    return loss_l_i, loss_c_i, npos


# speedup vs baseline: 12.5181x; 2.0228x over previous
import functools

import jax
import jax.numpy as jnp
from jax.experimental import pallas as pl
from jax.experimental.pallas import tpu as pltpu

_VARIANCES = (0.1, 0.2)
_THRESHOLD = 0.5
_NEGPOS_RATIO = 3

_NP = 8732
_R, _C = 72, 128
_PAD = _R * _C - _NP
_NB = 9
_BR = _R // _NB


def _body(truths_ref, labels_ref, priors_ref, loc_ref, conf_ref, out_ref,
          vA, vB, btov, btidx, pmax, pminidx, bp, part,
          num, num_objs, num_classes):
    i = pl.program_id(0)

    @pl.when(i < _NB)
    def _phase1():
        sl = i
        x = conf_ref[...]
        m = jnp.max(x, axis=1)
        s = jnp.sum(jnp.exp(x - m[:, None]), axis=1)
        lse = m + jnp.log(s)
        vA[:, pl.ds(sl * _BR, _BR), :] = lse - x[:, 0]
        vB[:, pl.ds(sl * _BR, _BR), :] = lse - x[:, 1]

        pcx = priors_ref[0]
        pcy = priors_ref[1]
        pw = priors_ref[2]
        ph = priors_ref[3]
        px1 = pcx - pw / 2.0
        py1 = pcy - ph / 2.0
        px2 = pcx + pw / 2.0
        py2 = pcy + ph / 2.0
        parea = (px2 - px1) * (py2 - py1)

        idx_blk = (sl * _BR * _C
                   + jax.lax.broadcasted_iota(jnp.int32, (_BR, _C), 0) * _C
                   + jax.lax.broadcasted_iota(jnp.int32, (_BR, _C), 1))

        bt_ov = jnp.full((num, _BR, _C), -1.0, dtype=jnp.float32)
        bt_id = jnp.zeros((num, _BR, _C), dtype=jnp.int32)
        for j in range(num_objs):
            tx1 = truths_ref[:, 0, j][:, None, None]
            ty1 = truths_ref[:, 1, j][:, None, None]
            tx2 = truths_ref[:, 2, j][:, None, None]
            ty2 = truths_ref[:, 3, j][:, None, None]
            ix = jnp.clip(jnp.minimum(px2, tx2) - jnp.maximum(px1, tx1),
                          0.0, None)
            iy = jnp.clip(jnp.minimum(py2, ty2) - jnp.maximum(py1, ty1),
                          0.0, None)
            inter = ix * iy
            tarea = (tx2 - tx1) * (ty2 - ty1)
            ov = inter / (tarea + parea - inter)
            upd = ov > bt_ov
            bt_ov = jnp.where(upd, ov, bt_ov)
            bt_id = jnp.where(upd, j, bt_id)
            mj = jnp.max(ov, axis=(1, 2))
            mn = jnp.min(jnp.where(ov == mj[:, None, None], idx_blk,
                                   jnp.int32(2**30)), axis=(1, 2))
            pmax[sl, j, :] = mj
            pminidx[sl, j, :] = mn
        btov[:, pl.ds(sl * _BR, _BR), :] = bt_ov
        btidx[:, pl.ds(sl * _BR, _BR), :] = bt_id

    @pl.when((i >= _NB) & (i < 2 * _NB))
    def _phase2():
        sl = i - _NB

        @pl.when(sl == 0)
        def _finalize_bp():
            pm = pmax[...]
            pi = pminidx[...]
            gmax = jnp.max(pm, axis=0)
            bp[...] = jnp.min(jnp.where(pm == gmax[None], pi,
                                        jnp.int32(2**30)), axis=0)

        pcx = priors_ref[0]
        pcy = priors_ref[1]
        pw = priors_ref[2]
        ph = priors_ref[3]

        idx_blk = (sl * _BR * _C
                   + jax.lax.broadcasted_iota(jnp.int32, (_BR, _C), 0) * _C
                   + jax.lax.broadcasted_iota(jnp.int32, (_BR, _C), 1))

        bt_ov = btov[:, pl.ds(sl * _BR, _BR), :]
        bt_id = btidx[:, pl.ds(sl * _BR, _BR), :]
        for j in range(num_objs):
            eq = idx_blk[None] == bp[j, :][:, None, None]
            bt_ov = jnp.where(eq, 2.0, bt_ov)
            bt_id = jnp.where(eq, j, bt_id)

        conf_t = jnp.zeros((num, _BR, _C), dtype=jnp.float32)
        for j in range(num_objs):
            conf_t = jnp.where(bt_id == j, labels_ref[:, j][:, None, None],
                               conf_t)
        conf_t = jnp.where(bt_ov < _THRESHOLD, 0.0, conf_t)
        pos = (conf_t > 0.0) & (idx_blk[None] < _NP)
        part[0, sl, :] = jnp.sum(pos.astype(jnp.float32), axis=(1, 2))

        mt = []
        for c in range(4):
            acc = jnp.zeros((num, _BR, _C), dtype=jnp.float32)
            for j in range(num_objs):
                acc = jnp.where(bt_id == j, truths_ref[:, c, j][:, None, None],
                                acc)
            mt.append(acc)
        mx1, my1, mx2, my2 = mt
        g = (((mx1 + mx2) / 2.0 - pcx) / (_VARIANCES[0] * pw),
             ((my1 + my2) / 2.0 - pcy) / (_VARIANCES[0] * ph),
             jnp.log((mx2 - mx1) / pw) / _VARIANCES[1],
             jnp.log((my2 - my1) / ph) / _VARIANCES[1])
        ll = jnp.zeros((num,), dtype=jnp.float32)
        for c in range(4):
            d = loc_ref[:, c] - g[c]
            ad = jnp.abs(d)
            sl1 = jnp.where(ad < 1.0, 0.5 * d * d, ad - 0.5)
            ll += jnp.sum(jnp.where(pos, sl1, 0.0), axis=(1, 2))
        part[1, sl, :] = ll

        vb_blk = vB[:, pl.ds(sl * _BR, _BR), :]
        part[2, sl, :] = jnp.sum(jnp.where(pos, vb_blk, 0.0), axis=(1, 2))
        va_blk = vA[:, pl.ds(sl * _BR, _BR), :]
        vA[:, pl.ds(sl * _BR, _BR), :] = jnp.maximum(
            jnp.where(pos | (idx_blk[None] >= _NP), 0.0, va_blk), 0.0)

    @pl.when(i == 2 * _NB)
    def _phase3():
        npos = jnp.sum(part[0], axis=0)
        ll_tot = jnp.sum(part[1])
        ce_pos = jnp.sum(part[2], axis=0)

        k = jnp.minimum((_NEGPOS_RATIO * npos).astype(jnp.int32),
                        jnp.int32(_NP - 1))
        v = vA[...]
        vb = jax.lax.bitcast_convert_type(v, jnp.int32)
        t = jnp.zeros((num,), dtype=jnp.int32)
        for b in range(30, -1, -1):
            cand = t | jnp.int32(1 << b)
            cnt = jnp.sum((vb >= cand[:, None, None]).astype(jnp.int32),
                          axis=(1, 2))
            t = jnp.where(cnt >= k, cand, t)
        cnt_gt = jnp.sum((vb > t[:, None, None]).astype(jnp.int32),
                         axis=(1, 2))
        sum_gt = jnp.sum(jnp.where(vb > t[:, None, None], v, 0.0),
                         axis=(1, 2))
        tval = jax.lax.bitcast_convert_type(t, jnp.float32)
        topk = sum_gt + (k - cnt_gt).astype(jnp.float32) * tval
        topk = jnp.where(k > 0, topk, 0.0)
        lc_tot = jnp.sum(ce_pos + topk)
        n = jnp.sum(npos)

        lane = jax.lax.broadcasted_iota(jnp.int32, (1, 128), 1)
        out_ref[...] = (jnp.where(lane == 0, ll_tot / n, 0.0)
                        + jnp.where(lane == 1, lc_tot / n, 0.0))


@jax.jit
def kernel(loc_data, conf_data, priors, targets):
    num, num_priors, num_classes = conf_data.shape
    num_objs = targets.shape[1]

    conf_p = jnp.pad(jnp.transpose(conf_data, (0, 2, 1)),
                     ((0, 0), (0, 0), (0, _PAD))).reshape(
                         num, num_classes, _R, _C)
    loc_p = jnp.pad(jnp.transpose(loc_data, (0, 2, 1)),
                    ((0, 0), (0, 0), (0, _PAD))).reshape(num, 4, _R, _C)
    priors_p = jnp.pad(priors.T, ((0, 0), (0, _PAD))).reshape(4, _R, _C)
    truths = jnp.transpose(targets[:, :, :4], (0, 2, 1))
    labels = targets[:, :, 4]

    body = functools.partial(_body, num=num, num_objs=num_objs,
                             num_classes=num_classes)
    out = pl.pallas_call(
        body,
        grid=(2 * _NB + 1,),
        in_specs=[
            pl.BlockSpec((num, 4, num_objs), lambda i: (0, 0, 0)),
            pl.BlockSpec((num, num_objs), lambda i: (0, 0)),
            pl.BlockSpec((4, _BR, _C),
                         lambda i: (0, jnp.where(i < _NB, i,
                                                 jnp.clip(i - _NB, 0, _NB - 1)),
                                    0)),
            pl.BlockSpec((num, 4, _BR, _C),
                         lambda i: (0, 0, jnp.clip(i - _NB, 0, _NB - 1), 0)),
            pl.BlockSpec((num, num_classes, _BR, _C),
                         lambda i: (0, 0, jnp.clip(i, 0, _NB - 1), 0)),
        ],
        out_specs=pl.BlockSpec((1, 128), lambda i: (0, 0)),
        out_shape=jax.ShapeDtypeStruct((1, 128), jnp.float32),
        scratch_shapes=[
            pltpu.VMEM((num, _R, _C), jnp.float32),
            pltpu.VMEM((num, _R, _C), jnp.float32),
            pltpu.VMEM((num, _R, _C), jnp.float32),
            pltpu.VMEM((num, _R, _C), jnp.int32),
            pltpu.VMEM((_NB, num_objs, num), jnp.float32),
            pltpu.VMEM((_NB, num_objs, num), jnp.int32),
            pltpu.VMEM((num_objs, num), jnp.int32),
            pltpu.VMEM((3, _NB, num), jnp.float32),
        ],
    )(truths, labels, priors_p, loc_p, conf_p)
    return (out[0, 0], out[0, 1])
